# Initial kernel scaffold; baseline (speedup 1.0000x reference)
import jax
import jax.numpy as jnp
from jax.experimental import pallas as pl


def kernel(xyz1, xyz2, points1, points2, W1, b1, W2, b2):
    raise NotImplementedError("write your pallas kernel here")


# speedup vs baseline: 28.3095x; 28.3095x over previous
import jax
import jax.numpy as jnp
from jax.experimental import pallas as pl

B, N, S = 8, 4096, 1024
D1, D2 = 128, 256
H1, H2 = 256, 128
NT = 512


def _fp_kernel(x1_ref, x2_ref, p1_ref, p2_ref, w1_ref, b1_ref, w2_ref, b2_ref,
               out_ref):
    x1 = x1_ref[0]
    x2 = x2_ref[0]
    sqn1 = jnp.sum(x1 * x1, axis=1, keepdims=True)
    sqn2 = jnp.sum(x2 * x2, axis=0, keepdims=True)
    dot = jnp.dot(x1, x2, preferred_element_type=jnp.float32)
    dist = sqn1 + sqn2 - 2.0 * dot

    lane = jax.lax.broadcasted_iota(jnp.int32, (NT, S), 1)
    big = jnp.float32(jnp.inf)

    def take_min(d):
        m = jnp.min(d, axis=1, keepdims=True)
        i = jnp.min(jnp.where(d == m, lane, S), axis=1, keepdims=True)
        d = jnp.where(lane == i, big, d)
        return m, i, d

    m1, i1, dist = take_min(dist)
    m2, i2, dist = take_min(dist)
    m3, i3, _ = take_min(dist)

    r1 = 1.0 / (m1 + 1e-8)
    r2 = 1.0 / (m2 + 1e-8)
    r3 = 1.0 / (m3 + 1e-8)
    norm = r1 + r2 + r3
    w1 = r1 / norm
    w2 = r2 / norm
    w3 = r3 / norm

    lane_t = jax.lax.broadcasted_iota(jnp.int32, (S, NT), 0)
    i1t = jnp.reshape(i1, (1, NT))
    i2t = jnp.reshape(i2, (1, NT))
    i3t = jnp.reshape(i3, (1, NT))
    zero = jnp.zeros((S, NT), jnp.float32)
    O = jnp.where(lane_t == i1t, jnp.reshape(w1, (1, NT)), zero)
    O = O + jnp.where(lane_t == i2t, jnp.reshape(w2, (1, NT)), zero)
    O = O + jnp.where(lane_t == i3t, jnp.reshape(w3, (1, NT)), zero)

    p2 = p2_ref[0]
    interp = jnp.dot(p2, O, preferred_element_type=jnp.float32)

    p1 = p1_ref[0]
    w1a = w1_ref[:, :D1]
    w1b = w1_ref[:, D1:]
    h = (jnp.dot(w1a, p1, preferred_element_type=jnp.float32)
         + jnp.dot(w1b, interp, preferred_element_type=jnp.float32)
         + b1_ref[:, :1])
    h = jnp.maximum(h, 0.0)
    out = jnp.dot(w2_ref[...], h, preferred_element_type=jnp.float32) + b2_ref[:, :1]
    out_ref[0] = jnp.maximum(out, 0.0)


def kernel(xyz1, xyz2, points1, points2, W1, b1, W2, b2):
    x1t = jnp.transpose(xyz1, (0, 2, 1))
    x1t = jnp.pad(x1t, ((0, 0), (0, 0), (0, 5)))
    x2p = jnp.pad(xyz2, ((0, 0), (0, 5), (0, 0)))
    b1c = jnp.reshape(b1, (H1, 1))
    b2c = jnp.reshape(b2, (H2, 1))

    grid = (B, N // NT)
    out = pl.pallas_call(
        _fp_kernel,
        grid=grid,
        in_specs=[
            pl.BlockSpec((1, NT, 8), lambda b, n: (b, n, 0)),
            pl.BlockSpec((1, 8, S), lambda b, n: (b, 0, 0)),
            pl.BlockSpec((1, D1, NT), lambda b, n: (b, 0, n)),
            pl.BlockSpec((1, D2, S), lambda b, n: (b, 0, 0)),
            pl.BlockSpec((H1, D1 + D2), lambda b, n: (0, 0)),
            pl.BlockSpec((H1, 1), lambda b, n: (0, 0)),
            pl.BlockSpec((H2, H1), lambda b, n: (0, 0)),
            pl.BlockSpec((H2, 1), lambda b, n: (0, 0)),
        ],
        out_specs=pl.BlockSpec((1, H2, NT), lambda b, n: (b, 0, n)),
        out_shape=jax.ShapeDtypeStruct((B, H2, N), jnp.float32),
    )(x1t, x2p, points1, points2, W1, b1c, W2, b2c)
    return out


# speedup vs baseline: 43.2370x; 1.5273x over previous
import jax
import jax.numpy as jnp
from jax.experimental import pallas as pl

B, N, S = 8, 4096, 1024
D1, D2 = 128, 256
H1, H2 = 256, 128
NT = 512


def _fp_kernel(x1_ref, x2_ref, p1_ref, p2_ref, w1_ref, b1_ref, w2_ref, b2_ref,
               out_ref):
    x1 = x1_ref[0]
    x2 = x2_ref[0]
    sqn1 = jnp.sum(x1 * x1, axis=0, keepdims=True)
    sqn2 = jnp.sum(x2 * x2, axis=1, keepdims=True)
    dot = jnp.dot(x2, x1, preferred_element_type=jnp.float32)
    dist = sqn2 + sqn1 - 2.0 * dot

    big = jnp.float32(jnp.inf)

    def take_min(d):
        m = jnp.min(d, axis=0, keepdims=True)
        msk = d == m
        d = jnp.where(msk, big, d)
        return m, msk, d

    m1, k1, dist = take_min(dist)
    m2, k2, dist = take_min(dist)
    m3, k3, dist = take_min(dist)

    r1 = 1.0 / (m1 + 1e-8)
    r2 = 1.0 / (m2 + 1e-8)
    r3 = 1.0 / (m3 + 1e-8)
    norm = r1 + r2 + r3
    w1 = r1 / norm
    w2 = r2 / norm
    w3 = r3 / norm

    zero = jnp.zeros((S, NT), jnp.float32)
    O = (jnp.where(k1, w1, zero) + jnp.where(k2, w2, zero)
         + jnp.where(k3, w3, zero))

    p2 = p2_ref[0]
    interp = jnp.dot(p2, O, preferred_element_type=jnp.float32)

    p1 = p1_ref[0]
    w1a = w1_ref[:, :D1]
    w1b = w1_ref[:, D1:]
    h = (jnp.dot(w1a, p1, preferred_element_type=jnp.float32)
         + jnp.dot(w1b, interp, preferred_element_type=jnp.float32)
         + b1_ref[:, :1])
    h = jnp.maximum(h, 0.0)
    out = jnp.dot(w2_ref[...], h, preferred_element_type=jnp.float32) + b2_ref[:, :1]
    out_ref[0] = jnp.maximum(out, 0.0)


def kernel(xyz1, xyz2, points1, points2, W1, b1, W2, b2):
    x1p = jnp.pad(xyz1, ((0, 0), (0, 5), (0, 0)))
    x2t = jnp.transpose(xyz2, (0, 2, 1))
    x2t = jnp.pad(x2t, ((0, 0), (0, 0), (0, 5)))
    b1c = jnp.reshape(b1, (H1, 1))
    b2c = jnp.reshape(b2, (H2, 1))

    grid = (B, N // NT)
    out = pl.pallas_call(
        _fp_kernel,
        grid=grid,
        in_specs=[
            pl.BlockSpec((1, 8, NT), lambda b, n: (b, 0, n)),
            pl.BlockSpec((1, S, 8), lambda b, n: (b, 0, 0)),
            pl.BlockSpec((1, D1, NT), lambda b, n: (b, 0, n)),
            pl.BlockSpec((1, D2, S), lambda b, n: (b, 0, 0)),
            pl.BlockSpec((H1, D1 + D2), lambda b, n: (0, 0)),
            pl.BlockSpec((H1, 1), lambda b, n: (0, 0)),
            pl.BlockSpec((H2, H1), lambda b, n: (0, 0)),
            pl.BlockSpec((H2, 1), lambda b, n: (0, 0)),
        ],
        out_specs=pl.BlockSpec((1, H2, NT), lambda b, n: (b, 0, n)),
        out_shape=jax.ShapeDtypeStruct((B, H2, N), jnp.float32),
    )(x1p, x2t, points1, points2, W1, b1c, W2, b2c)
    return out
